# Initial kernel scaffold; baseline (speedup 1.0000x reference)
import jax
import jax.numpy as jnp
from jax.experimental import pallas as pl


def kernel(embeddings, edge_index, W1, b1, W2, b2):
    raise NotImplementedError("write your pallas kernel here")


# speedup vs baseline: 13.2580x; 13.2580x over previous
import functools

import jax
import jax.numpy as jnp
from jax import lax
from jax.experimental import pallas as pl
from jax.experimental.pallas import tpu as pltpu
from jax.experimental.pallas import tpu_sc as plsc

_NC, _NS, _L = 2, 16, 16
_NW = _NC * _NS
_C = 80
_DW = 16


def _pad_rows(N):
    per = ((N + _NS - 1) // _NS + 7) // 8 * 8
    return per * _NS, per


@functools.lru_cache(maxsize=None)
def _deg_kernel(N, E):
    epw = E // _NW
    nchunk = epw // _C
    npad, rps = _pad_rows(N)
    zr = 128
    mesh = plsc.VectorSubcoreMesh(core_axis_name="c", subcore_axis_name="s",
                                  num_cores=_NC, num_subcores=_NS)

    @functools.partial(
        pl.kernel,
        out_type=jax.ShapeDtypeStruct((_NC, npad, _DW), jnp.float32),
        mesh=mesh,
        scratch_types=[
            pltpu.VMEM((_C,), jnp.int32),
            pltpu.VMEM((_C, _DW), jnp.float32),
            pltpu.VMEM((zr, _DW), jnp.float32),
            pltpu.VMEM_SHARED((npad, _DW), jnp.float32),
        ],
    )
    def deg_k(dst_hbm, out_hbm, didx, ones, zbuf, acc):
        cid = lax.axis_index("c")
        sid = lax.axis_index("s")
        wid = sid * _NC + cid
        zv = jnp.zeros((_L,), jnp.float32)
        ov = jnp.ones((_L,), jnp.float32)

        def fill_ones(i, _):
            ones[i, :] = ov
            return 0
        lax.fori_loop(0, _C, fill_ones, 0)

        def fill_zero(i, _):
            zbuf[i, :] = zv
            return 0
        lax.fori_loop(0, zr, fill_zero, 0)

        def zero_acc(i, _):
            pltpu.sync_copy(zbuf, acc.at[pl.ds(sid * rps + i * zr, zr)])
            return 0
        lax.fori_loop(0, rps // zr, zero_acc, 0)
        plsc.subcore_barrier()

        ebase = wid * epw

        def body(j, _):
            pltpu.sync_copy(dst_hbm.at[pl.ds(ebase + j * _C, _C)], didx)
            pltpu.sync_copy(ones, acc.at[didx], add=True)
            return 0
        lax.fori_loop(0, nchunk, body, 0)
        plsc.subcore_barrier()

        pltpu.sync_copy(acc.at[pl.ds(sid * rps, rps)],
                        out_hbm.at[cid, pl.ds(sid * rps, rps)])

    return deg_k


@functools.lru_cache(maxsize=None)
def _agg_kernel(N, E, H):
    epw = E // _NW
    nchunk = epw // _C
    npad, rps = _pad_rows(N)
    zr = 80
    mesh = plsc.VectorSubcoreMesh(core_axis_name="c", subcore_axis_name="s",
                                  num_cores=_NC, num_subcores=_NS)

    @functools.partial(
        pl.kernel,
        out_type=jax.ShapeDtypeStruct((_NC, npad, H), jnp.float32),
        mesh=mesh,
        scratch_types=[
            pltpu.VMEM((_C,), jnp.int32),
            pltpu.VMEM((_C,), jnp.int32),
            pltpu.VMEM((_C, H), jnp.float32),
            pltpu.VMEM((zr, H), jnp.float32),
            pltpu.VMEM_SHARED((npad, H), jnp.float32),
            pltpu.SemaphoreType.DMA,
        ],
    )
    def agg_k(y_hbm, src_hbm, dst_hbm, out_hbm, sidx, didx, rows, zbuf, acc, sem):
        cid = lax.axis_index("c")
        sid = lax.axis_index("s")
        wid = sid * _NC + cid
        zv = jnp.zeros((_L,), jnp.float32)

        def fill_zero(i, _):
            for k in range(H // _L):
                zbuf[i, pl.ds(k * _L, _L)] = zv
            return 0
        lax.fori_loop(0, zr, fill_zero, 0)

        def zero_acc(i, _):
            pltpu.sync_copy(zbuf, acc.at[pl.ds(sid * rps + i * zr, zr)])
            return 0
        lax.fori_loop(0, rps // zr, zero_acc, 0)
        plsc.subcore_barrier()

        ebase = wid * epw

        def body(j, _):
            pltpu.sync_copy(src_hbm.at[pl.ds(ebase + j * _C, _C)], sidx)
            pltpu.sync_copy(dst_hbm.at[pl.ds(ebase + j * _C, _C)], didx)
            pltpu.async_copy(y_hbm.at[sidx], rows, sem).wait()
            pltpu.sync_copy(rows, acc.at[didx], add=True)
            return 0
        lax.fori_loop(0, nchunk, body, 0)
        plsc.subcore_barrier()

        pltpu.sync_copy(acc.at[pl.ds(sid * rps, rps)],
                        out_hbm.at[cid, pl.ds(sid * rps, rps)])

    return agg_k


def _dinv_block(dp_ref):
    deg = dp_ref[0, :, 0:1] + dp_ref[1, :, 0:1] + 1.0
    return lax.rsqrt(deg)


def _prep_call(deg_parts, x, W):
    N, H = x.shape
    bn = 2000

    def body(dp_ref, x_ref, w_ref, y_ref):
        dinv = _dinv_block(dp_ref)
        z = jnp.dot(x_ref[...], w_ref[...], preferred_element_type=jnp.float32)
        y_ref[...] = z * dinv

    return pl.pallas_call(
        body,
        grid=(N // bn,),
        in_specs=[
            pl.BlockSpec((_NC, bn, _DW), lambda i: (0, i, 0)),
            pl.BlockSpec((bn, H), lambda i: (i, 0)),
            pl.BlockSpec((H, H), lambda i: (0, 0)),
        ],
        out_specs=pl.BlockSpec((bn, H), lambda i: (i, 0)),
        out_shape=jax.ShapeDtypeStruct((N, H), jnp.float32),
    )(deg_parts, x, W)


def _mid_call(deg_parts, agg, y, b, W):
    N, H = y.shape
    bn = 2000

    def body(dp_ref, a_ref, y_ref, b_ref, w_ref, o_ref):
        dinv = _dinv_block(dp_ref)
        h = (a_ref[0] + a_ref[1] + y_ref[...]) * dinv + b_ref[...]
        o_ref[...] = jnp.dot(h, w_ref[...],
                             preferred_element_type=jnp.float32) * dinv

    return pl.pallas_call(
        body,
        grid=(N // bn,),
        in_specs=[
            pl.BlockSpec((_NC, bn, _DW), lambda i: (0, i, 0)),
            pl.BlockSpec((_NC, bn, H), lambda i: (0, i, 0)),
            pl.BlockSpec((bn, H), lambda i: (i, 0)),
            pl.BlockSpec((1, H), lambda i: (0, 0)),
            pl.BlockSpec((H, H), lambda i: (0, 0)),
        ],
        out_specs=pl.BlockSpec((bn, H), lambda i: (i, 0)),
        out_shape=jax.ShapeDtypeStruct((N, H), jnp.float32),
    )(deg_parts, agg, y, b, W)


def _fin_call(deg_parts, agg, y, b):
    N, H = y.shape
    bn = 2000

    def body(dp_ref, a_ref, y_ref, b_ref, o_ref):
        dinv = _dinv_block(dp_ref)
        o_ref[...] = (a_ref[0] + a_ref[1] + y_ref[...]) * dinv + b_ref[...]

    return pl.pallas_call(
        body,
        grid=(N // bn,),
        in_specs=[
            pl.BlockSpec((_NC, bn, _DW), lambda i: (0, i, 0)),
            pl.BlockSpec((_NC, bn, H), lambda i: (0, i, 0)),
            pl.BlockSpec((bn, H), lambda i: (i, 0)),
            pl.BlockSpec((1, H), lambda i: (0, 0)),
        ],
        out_specs=pl.BlockSpec((bn, H), lambda i: (i, 0)),
        out_shape=jax.ShapeDtypeStruct((N, H), jnp.float32),
    )(deg_parts, agg, y, b)


def kernel(embeddings, edge_index, W1, b1, W2, b2):
    x = embeddings
    src = edge_index[0].astype(jnp.int32)
    dst = edge_index[1].astype(jnp.int32)
    N, H = x.shape
    E = src.shape[0]

    deg_parts = _deg_kernel(N, E)(dst)
    y1 = _prep_call(deg_parts, x, W1)
    agg1 = _agg_kernel(N, E, H)(y1, src, dst)
    y2 = _mid_call(deg_parts, agg1, y1, b1.reshape(1, H), W2)
    agg2 = _agg_kernel(N, E, H)(y2, src, dst)
    return _fin_call(deg_parts, agg2, y2, b2.reshape(1, H))


# speedup vs baseline: 29.6979x; 2.2400x over previous
import functools

import jax
import jax.numpy as jnp
from jax import lax
from jax.experimental import pallas as pl
from jax.experimental.pallas import tpu as pltpu
from jax.experimental.pallas import tpu_sc as plsc

_NC, _NS, _L = 2, 16, 16
_NW = _NC * _NS
_C = 80
_DW = 16


def _pad_rows(N):
    per = ((N + _NS - 1) // _NS + 7) // 8 * 8
    return per * _NS, per


@functools.lru_cache(maxsize=None)
def _deg_kernel(N, E):
    epw = E // _NW
    nchunk = epw // _C
    npad, rps = _pad_rows(N)
    zr = 128
    mesh = plsc.VectorSubcoreMesh(core_axis_name="c", subcore_axis_name="s",
                                  num_cores=_NC, num_subcores=_NS)

    assert nchunk % 2 == 1

    @functools.partial(
        pl.kernel,
        out_type=jax.ShapeDtypeStruct((_NC, npad, _DW), jnp.float32),
        mesh=mesh,
        scratch_types=[
            pltpu.VMEM((_C,), jnp.int32),
            pltpu.VMEM((_C,), jnp.int32),
            pltpu.VMEM((_C, _DW), jnp.float32),
            pltpu.VMEM((zr, _DW), jnp.float32),
            pltpu.VMEM_SHARED((npad, _DW), jnp.float32),
            pltpu.SemaphoreType.DMA,
            pltpu.SemaphoreType.DMA,
        ],
    )
    def deg_k(dst_hbm, out_hbm, didx0, didx1, ones, zbuf, acc, si0, si1):
        cid = lax.axis_index("c")
        sid = lax.axis_index("s")
        wid = sid * _NC + cid
        zv = jnp.zeros((_L,), jnp.float32)
        ov = jnp.ones((_L,), jnp.float32)
        didx = (didx0, didx1)
        si = (si0, si1)

        def fill_ones(i, _):
            ones[i, :] = ov
            return 0
        lax.fori_loop(0, _C, fill_ones, 0)

        def fill_zero(i, _):
            zbuf[i, :] = zv
            return 0
        lax.fori_loop(0, zr, fill_zero, 0)

        def zero_acc(i, _):
            pltpu.sync_copy(zbuf, acc.at[pl.ds(sid * rps + i * zr, zr)])
            return 0
        lax.fori_loop(0, rps // zr, zero_acc, 0)
        plsc.subcore_barrier()

        ebase = wid * epw

        def start(j, b):
            pltpu.async_copy(dst_hbm.at[pl.ds(ebase + j * _C, _C)],
                             didx[b], si[b])

        def finish(b):
            pltpu.make_async_copy(dst_hbm.at[pl.ds(0, _C)],
                                  didx[b], si[b]).wait()
            pltpu.sync_copy(ones, acc.at[didx[b]], add=True)

        start(0, 0)

        def body(jj, _):
            j0 = 2 * jj
            start(j0 + 1, 1)
            finish(0)
            start(j0 + 2, 0)
            finish(1)
            return 0
        lax.fori_loop(0, nchunk // 2, body, 0)
        finish(0)
        plsc.subcore_barrier()

        pltpu.sync_copy(acc.at[pl.ds(sid * rps, rps)],
                        out_hbm.at[cid, pl.ds(sid * rps, rps)])

    return deg_k


@functools.lru_cache(maxsize=None)
def _agg_kernel(N, E, H):
    epw = E // _NW
    nchunk = epw // _C
    npad, rps = _pad_rows(N)
    zr = 80
    mesh = plsc.VectorSubcoreMesh(core_axis_name="c", subcore_axis_name="s",
                                  num_cores=_NC, num_subcores=_NS)

    assert nchunk % 2 == 1

    @functools.partial(
        pl.kernel,
        out_type=jax.ShapeDtypeStruct((_NC, npad, H), jnp.float32),
        mesh=mesh,
        scratch_types=[
            pltpu.VMEM((epw,), jnp.int32),
            pltpu.VMEM((_C,), jnp.int32),
            pltpu.VMEM((_C,), jnp.int32),
            pltpu.VMEM((_C, H), jnp.float32),
            pltpu.VMEM((_C, H), jnp.float32),
            pltpu.VMEM((zr, H), jnp.float32),
            pltpu.VMEM_SHARED((npad, H), jnp.float32),
            pltpu.SemaphoreType.DMA,
            pltpu.SemaphoreType.DMA,
            pltpu.SemaphoreType.DMA,
            pltpu.SemaphoreType.DMA,
        ],
    )
    def agg_k(y_hbm, src_hbm, dst_hbm, out_hbm, sidx_all, didx0, didx1,
              rows0, rows1, zbuf, acc, si0, si1, sg0, sg1):
        cid = lax.axis_index("c")
        sid = lax.axis_index("s")
        wid = sid * _NC + cid
        zv = jnp.zeros((_L,), jnp.float32)
        didx = (didx0, didx1)
        rows = (rows0, rows1)
        si = (si0, si1)
        sg = (sg0, sg1)

        ebase = wid * epw
        pltpu.async_copy(src_hbm.at[pl.ds(ebase, epw)], sidx_all, si0)

        def fill_zero(i, _):
            for k in range(H // _L):
                zbuf[i, pl.ds(k * _L, _L)] = zv
            return 0
        lax.fori_loop(0, zr, fill_zero, 0)

        def zero_acc(i, _):
            pltpu.sync_copy(zbuf, acc.at[pl.ds(sid * rps + i * zr, zr)])
            return 0
        lax.fori_loop(0, rps // zr, zero_acc, 0)
        pltpu.make_async_copy(src_hbm.at[pl.ds(0, epw)], sidx_all, si0).wait()
        plsc.subcore_barrier()

        def start(j, b):
            pltpu.async_copy(dst_hbm.at[pl.ds(ebase + j * _C, _C)],
                             didx[b], si[b])
            pltpu.async_copy(y_hbm.at[sidx_all.at[pl.ds(j * _C, _C)]],
                             rows[b], sg[b])

        def finish(b):
            pltpu.make_async_copy(dst_hbm.at[pl.ds(0, _C)],
                                  didx[b], si[b]).wait()
            pltpu.make_async_copy(y_hbm.at[pl.ds(0, _C)], rows[b],
                                  sg[b]).wait()
            pltpu.sync_copy(rows[b], acc.at[didx[b]], add=True)

        start(0, 0)

        def body(jj, _):
            j0 = 2 * jj
            start(j0 + 1, 1)
            finish(0)
            start(j0 + 2, 0)
            finish(1)
            return 0
        lax.fori_loop(0, nchunk // 2, body, 0)
        finish(0)
        plsc.subcore_barrier()

        pltpu.sync_copy(acc.at[pl.ds(sid * rps, rps)],
                        out_hbm.at[cid, pl.ds(sid * rps, rps)])

    return agg_k


def _dinv_block(dp_ref):
    deg = dp_ref[0, :, 0:1] + dp_ref[1, :, 0:1] + 1.0
    return lax.rsqrt(deg)


def _prep_call(deg_parts, x, W):
    N, H = x.shape
    bn = 2000

    def body(dp_ref, x_ref, w_ref, y_ref):
        dinv = _dinv_block(dp_ref)
        z = jnp.dot(x_ref[...], w_ref[...], preferred_element_type=jnp.float32)
        y_ref[...] = z * dinv

    return pl.pallas_call(
        body,
        grid=(N // bn,),
        in_specs=[
            pl.BlockSpec((_NC, bn, _DW), lambda i: (0, i, 0)),
            pl.BlockSpec((bn, H), lambda i: (i, 0)),
            pl.BlockSpec((H, H), lambda i: (0, 0)),
        ],
        out_specs=pl.BlockSpec((bn, H), lambda i: (i, 0)),
        out_shape=jax.ShapeDtypeStruct((N, H), jnp.float32),
    )(deg_parts, x, W)


def _mid_call(deg_parts, agg, y, b, W):
    N, H = y.shape
    bn = 2000

    def body(dp_ref, a_ref, y_ref, b_ref, w_ref, o_ref):
        dinv = _dinv_block(dp_ref)
        h = (a_ref[0] + a_ref[1] + y_ref[...]) * dinv + b_ref[...]
        o_ref[...] = jnp.dot(h, w_ref[...],
                             preferred_element_type=jnp.float32) * dinv

    return pl.pallas_call(
        body,
        grid=(N // bn,),
        in_specs=[
            pl.BlockSpec((_NC, bn, _DW), lambda i: (0, i, 0)),
            pl.BlockSpec((_NC, bn, H), lambda i: (0, i, 0)),
            pl.BlockSpec((bn, H), lambda i: (i, 0)),
            pl.BlockSpec((1, H), lambda i: (0, 0)),
            pl.BlockSpec((H, H), lambda i: (0, 0)),
        ],
        out_specs=pl.BlockSpec((bn, H), lambda i: (i, 0)),
        out_shape=jax.ShapeDtypeStruct((N, H), jnp.float32),
    )(deg_parts, agg, y, b, W)


def _fin_call(deg_parts, agg, y, b):
    N, H = y.shape
    bn = 2000

    def body(dp_ref, a_ref, y_ref, b_ref, o_ref):
        dinv = _dinv_block(dp_ref)
        o_ref[...] = (a_ref[0] + a_ref[1] + y_ref[...]) * dinv + b_ref[...]

    return pl.pallas_call(
        body,
        grid=(N // bn,),
        in_specs=[
            pl.BlockSpec((_NC, bn, _DW), lambda i: (0, i, 0)),
            pl.BlockSpec((_NC, bn, H), lambda i: (0, i, 0)),
            pl.BlockSpec((bn, H), lambda i: (i, 0)),
            pl.BlockSpec((1, H), lambda i: (0, 0)),
        ],
        out_specs=pl.BlockSpec((bn, H), lambda i: (i, 0)),
        out_shape=jax.ShapeDtypeStruct((N, H), jnp.float32),
    )(deg_parts, agg, y, b)


def kernel(embeddings, edge_index, W1, b1, W2, b2):
    x = embeddings
    src = edge_index[0].astype(jnp.int32)
    dst = edge_index[1].astype(jnp.int32)
    N, H = x.shape
    E = src.shape[0]

    deg_parts = _deg_kernel(N, E)(dst)
    y1 = _prep_call(deg_parts, x, W1)
    agg1 = _agg_kernel(N, E, H)(y1, src, dst)
    y2 = _mid_call(deg_parts, agg1, y1, b1.reshape(1, H), W2)
    agg2 = _agg_kernel(N, E, H)(y2, src, dst)
    return _fin_call(deg_parts, agg2, y2, b2.reshape(1, H))


# speedup vs baseline: 32.6865x; 1.1006x over previous
import functools

import jax
import jax.numpy as jnp
from jax import lax
from jax.experimental import pallas as pl
from jax.experimental.pallas import tpu as pltpu
from jax.experimental.pallas import tpu_sc as plsc

_NC, _NS, _L = 2, 16, 16
_NW = _NC * _NS
_C = 128
_NB = 2
_DW = 16


def _pad_rows(N):
    per = ((N + _NS - 1) // _NS + 127) // 128 * 128
    return per * _NS, per


def _pipeline(nch, start, finish):
    for j in range(min(_NB - 1, nch)):
        start(j, j)

    def body(jj, _):
        for k in range(_NB):
            j = jj * _NB + k

            def do_start(js=j + _NB - 1, bs=(k + _NB - 1) % _NB):
                start(js, bs)

            def do_finish(jf=j, bf=k):
                finish(jf, bf)

            pl.when(j + _NB - 1 < nch)(do_start)
            pl.when(j < nch)(do_finish)
        return 0

    lax.fori_loop(0, (nch + _NB - 1) // _NB, body, 0)


@functools.lru_cache(maxsize=None)
def _deg_kernel(N, E):
    epw = E // _NW
    nch = epw // _C
    tail = epw - nch * _C
    npad, rps = _pad_rows(N)
    zr = 128
    assert rps % zr == 0 and tail % 8 == 0
    mesh = plsc.VectorSubcoreMesh(core_axis_name="c", subcore_axis_name="s",
                                  num_cores=_NC, num_subcores=_NS)

    @functools.partial(
        pl.kernel,
        out_type=jax.ShapeDtypeStruct((_NC, npad, _DW), jnp.float32),
        mesh=mesh,
        scratch_types=[
            [pltpu.VMEM((_C,), jnp.int32) for _ in range(_NB)],
            pltpu.VMEM((tail,), jnp.int32) if tail else None,
            pltpu.VMEM((_C, _DW), jnp.float32),
            pltpu.VMEM((zr, _DW), jnp.float32),
            pltpu.VMEM_SHARED((npad, _DW), jnp.float32),
            [pltpu.SemaphoreType.DMA for _ in range(_NB)],
            pltpu.SemaphoreType.DMA,
        ],
    )
    def deg_k(dst_hbm, out_hbm, didx, didx_t, ones, zbuf, acc, si, si_t):
        cid = lax.axis_index("c")
        sid = lax.axis_index("s")
        wid = sid * _NC + cid
        ebase = wid * epw
        zv = jnp.zeros((_L,), jnp.float32)
        ov = jnp.ones((_L,), jnp.float32)

        if tail:
            pltpu.async_copy(dst_hbm.at[pl.ds(ebase + nch * _C, tail)],
                             didx_t, si_t)

        def fill_ones(i, _):
            ones[i, :] = ov
            return 0
        lax.fori_loop(0, _C, fill_ones, 0)

        def fill_zero(i, _):
            zbuf[i, :] = zv
            return 0
        lax.fori_loop(0, zr, fill_zero, 0)

        def zero_acc(i, _):
            pltpu.sync_copy(zbuf, acc.at[pl.ds(sid * rps + i * zr, zr)])
            return 0
        lax.fori_loop(0, rps // zr, zero_acc, 0)
        plsc.subcore_barrier()

        def start(j, b):
            pltpu.async_copy(dst_hbm.at[pl.ds(ebase + j * _C, _C)],
                             didx[b], si[b])

        def finish(j, b):
            del j
            pltpu.make_async_copy(dst_hbm.at[pl.ds(0, _C)],
                                  didx[b], si[b]).wait()
            pltpu.sync_copy(ones, acc.at[didx[b]], add=True)

        _pipeline(nch, start, finish)
        if tail:
            pltpu.make_async_copy(dst_hbm.at[pl.ds(0, tail)],
                                  didx_t, si_t).wait()
            pltpu.sync_copy(ones.at[pl.ds(0, tail)], acc.at[didx_t], add=True)
        plsc.subcore_barrier()

        pltpu.sync_copy(acc.at[pl.ds(sid * rps, rps)],
                        out_hbm.at[cid, pl.ds(sid * rps, rps)])

    return deg_k


@functools.lru_cache(maxsize=None)
def _agg_kernel(N, E, H):
    epw = E // _NW
    nch = epw // _C
    tail = epw - nch * _C
    npad, rps = _pad_rows(N)
    zr = 32
    assert rps % zr == 0 and tail % 8 == 0 and epw % 8 == 0
    mesh = plsc.VectorSubcoreMesh(core_axis_name="c", subcore_axis_name="s",
                                  num_cores=_NC, num_subcores=_NS)

    @functools.partial(
        pl.kernel,
        out_type=jax.ShapeDtypeStruct((_NC, npad, H), jnp.float32),
        mesh=mesh,
        scratch_types=[
            pltpu.VMEM((epw,), jnp.int32),
            [pltpu.VMEM((_C,), jnp.int32) for _ in range(_NB)],
            pltpu.VMEM((tail,), jnp.int32) if tail else None,
            [pltpu.VMEM((_C, H), jnp.float32) for _ in range(_NB)],
            pltpu.VMEM((zr, H), jnp.float32),
            pltpu.VMEM_SHARED((npad, H), jnp.float32),
            [pltpu.SemaphoreType.DMA for _ in range(_NB)],
            [pltpu.SemaphoreType.DMA for _ in range(_NB)],
            pltpu.SemaphoreType.DMA,
        ],
    )
    def agg_k(y_hbm, src_hbm, dst_hbm, out_hbm, sidx_all, didx, didx_t,
              rows, zbuf, acc, si, sg, s_t):
        cid = lax.axis_index("c")
        sid = lax.axis_index("s")
        wid = sid * _NC + cid
        ebase = wid * epw
        zv = jnp.zeros((_L,), jnp.float32)

        pltpu.async_copy(src_hbm.at[pl.ds(ebase, epw)], sidx_all, s_t)
        if tail:
            pltpu.async_copy(dst_hbm.at[pl.ds(ebase + nch * _C, tail)],
                             didx_t, si[0])

        def fill_zero(i, _):
            for k in range(H // _L):
                zbuf[i, pl.ds(k * _L, _L)] = zv
            return 0
        lax.fori_loop(0, zr, fill_zero, 0)

        def zero_start(i, _):
            pltpu.async_copy(zbuf, acc.at[pl.ds(sid * rps + i * zr, zr)],
                             sg[0])
            return 0
        lax.fori_loop(0, rps // zr, zero_start, 0)

        def zero_wait(i, _):
            pltpu.make_async_copy(zbuf, acc.at[pl.ds(sid * rps + i * zr, zr)],
                                  sg[0]).wait()
            return 0
        lax.fori_loop(0, rps // zr, zero_wait, 0)
        pltpu.make_async_copy(src_hbm.at[pl.ds(0, epw)], sidx_all, s_t).wait()
        if tail:
            pltpu.make_async_copy(dst_hbm.at[pl.ds(0, tail)],
                                  didx_t, si[0]).wait()
        plsc.subcore_barrier()

        def start(j, b):
            pltpu.async_copy(dst_hbm.at[pl.ds(ebase + j * _C, _C)],
                             didx[b], si[b])
            pltpu.async_copy(y_hbm.at[sidx_all.at[pl.ds(j * _C, _C)]],
                             rows[b], sg[b])

        def finish(j, b):
            del j
            pltpu.make_async_copy(dst_hbm.at[pl.ds(0, _C)],
                                  didx[b], si[b]).wait()
            pltpu.make_async_copy(y_hbm.at[pl.ds(0, _C)], rows[b],
                                  sg[b]).wait()
            pltpu.sync_copy(rows[b], acc.at[didx[b]], add=True)

        _pipeline(nch, start, finish)
        if tail:
            rows_t = rows[0].at[pl.ds(0, tail)]
            pltpu.async_copy(y_hbm.at[sidx_all.at[pl.ds(nch * _C, tail)]],
                             rows_t, s_t)
            pltpu.make_async_copy(y_hbm.at[pl.ds(0, tail)], rows_t,
                                  s_t).wait()
            pltpu.sync_copy(rows_t, acc.at[didx_t], add=True)
        plsc.subcore_barrier()

        pltpu.sync_copy(acc.at[pl.ds(sid * rps, rps)],
                        out_hbm.at[cid, pl.ds(sid * rps, rps)])

    return agg_k


def _dinv_block(dp_ref):
    deg = dp_ref[0, :, 0:1] + dp_ref[1, :, 0:1] + 1.0
    return lax.rsqrt(deg)


def _prep_call(deg_parts, x, W):
    N, H = x.shape
    bn = 2000

    def body(dp_ref, x_ref, w_ref, y_ref):
        dinv = _dinv_block(dp_ref)
        z = jnp.dot(x_ref[...], w_ref[...], preferred_element_type=jnp.float32)
        y_ref[...] = z * dinv

    return pl.pallas_call(
        body,
        grid=(N // bn,),
        in_specs=[
            pl.BlockSpec((_NC, bn, _DW), lambda i: (0, i, 0)),
            pl.BlockSpec((bn, H), lambda i: (i, 0)),
            pl.BlockSpec((H, H), lambda i: (0, 0)),
        ],
        out_specs=pl.BlockSpec((bn, H), lambda i: (i, 0)),
        out_shape=jax.ShapeDtypeStruct((N, H), jnp.float32),
    )(deg_parts, x, W)


def _mid_call(deg_parts, agg, y, b, W):
    N, H = y.shape
    bn = 2000

    def body(dp_ref, a_ref, y_ref, b_ref, w_ref, o_ref):
        dinv = _dinv_block(dp_ref)
        h = (a_ref[0] + a_ref[1] + y_ref[...]) * dinv + b_ref[...]
        o_ref[...] = jnp.dot(h, w_ref[...],
                             preferred_element_type=jnp.float32) * dinv

    return pl.pallas_call(
        body,
        grid=(N // bn,),
        in_specs=[
            pl.BlockSpec((_NC, bn, _DW), lambda i: (0, i, 0)),
            pl.BlockSpec((_NC, bn, H), lambda i: (0, i, 0)),
            pl.BlockSpec((bn, H), lambda i: (i, 0)),
            pl.BlockSpec((1, H), lambda i: (0, 0)),
            pl.BlockSpec((H, H), lambda i: (0, 0)),
        ],
        out_specs=pl.BlockSpec((bn, H), lambda i: (i, 0)),
        out_shape=jax.ShapeDtypeStruct((N, H), jnp.float32),
    )(deg_parts, agg, y, b, W)


def _fin_call(deg_parts, agg, y, b):
    N, H = y.shape
    bn = 2000

    def body(dp_ref, a_ref, y_ref, b_ref, o_ref):
        dinv = _dinv_block(dp_ref)
        o_ref[...] = (a_ref[0] + a_ref[1] + y_ref[...]) * dinv + b_ref[...]

    return pl.pallas_call(
        body,
        grid=(N // bn,),
        in_specs=[
            pl.BlockSpec((_NC, bn, _DW), lambda i: (0, i, 0)),
            pl.BlockSpec((_NC, bn, H), lambda i: (0, i, 0)),
            pl.BlockSpec((bn, H), lambda i: (i, 0)),
            pl.BlockSpec((1, H), lambda i: (0, 0)),
        ],
        out_specs=pl.BlockSpec((bn, H), lambda i: (i, 0)),
        out_shape=jax.ShapeDtypeStruct((N, H), jnp.float32),
    )(deg_parts, agg, y, b)


def kernel(embeddings, edge_index, W1, b1, W2, b2):
    x = embeddings
    src = edge_index[0].astype(jnp.int32)
    dst = edge_index[1].astype(jnp.int32)
    N, H = x.shape
    E = src.shape[0]

    deg_parts = _deg_kernel(N, E)(dst)
    y1 = _prep_call(deg_parts, x, W1)
    agg1 = _agg_kernel(N, E, H)(y1, src, dst)
    y2 = _mid_call(deg_parts, agg1, y1, b1.reshape(1, H), W2)
    agg2 = _agg_kernel(N, E, H)(y2, src, dst)
    return _fin_call(deg_parts, agg2, y2, b2.reshape(1, H))


# speedup vs baseline: 34.0616x; 1.0421x over previous
import functools

import jax
import jax.numpy as jnp
from jax import lax
from jax.experimental import pallas as pl
from jax.experimental.pallas import tpu as pltpu
from jax.experimental.pallas import tpu_sc as plsc

_NC, _NS, _L = 2, 16, 16
_NW = _NC * _NS
_C = 128
_NB = 2
_DW = 16


def _pad_rows(N):
    per = ((N + _NS - 1) // _NS + 127) // 128 * 128
    return per * _NS, per


def _pipeline(nch, start, finish, nb):
    for j in range(min(nb - 1, nch)):
        start(j, j)

    def body(jj, _):
        for k in range(nb):
            j = jj * nb + k

            def do_start(js=j + nb - 1, bs=(k + nb - 1) % nb):
                start(js, bs)

            def do_finish(jf=j, bf=k):
                finish(jf, bf)

            pl.when(j + nb - 1 < nch)(do_start)
            pl.when(j < nch)(do_finish)
        return 0

    lax.fori_loop(0, (nch + nb - 1) // nb, body, 0)


@functools.lru_cache(maxsize=None)
def _deg_kernel(N, E):
    epw = E // _NW
    nch = epw // _C
    tail = epw - nch * _C
    npad, rps = _pad_rows(N)
    zr = 128
    assert rps % zr == 0 and tail % 8 == 0
    mesh = plsc.VectorSubcoreMesh(core_axis_name="c", subcore_axis_name="s",
                                  num_cores=_NC, num_subcores=_NS)

    nb = 4

    @functools.partial(
        pl.kernel,
        out_type=jax.ShapeDtypeStruct((_NC, npad, _DW), jnp.float32),
        mesh=mesh,
        scratch_types=[
            [pltpu.VMEM((_C,), jnp.int32) for _ in range(nb)],
            pltpu.VMEM((tail,), jnp.int32) if tail else None,
            pltpu.VMEM((_C, _DW), jnp.float32),
            pltpu.VMEM((zr, _DW), jnp.float32),
            pltpu.VMEM_SHARED((npad, _DW), jnp.float32),
            [pltpu.SemaphoreType.DMA for _ in range(nb)],
            [pltpu.SemaphoreType.DMA for _ in range(nb)],
            pltpu.SemaphoreType.DMA,
        ],
    )
    def deg_k(dst_hbm, out_hbm, didx, didx_t, ones, zbuf, acc, si, ss, si_t):
        cid = lax.axis_index("c")
        sid = lax.axis_index("s")
        wid = sid * _NC + cid
        ebase = wid * epw
        zv = jnp.zeros((_L,), jnp.float32)
        ov = jnp.ones((_L,), jnp.float32)

        if tail:
            pltpu.async_copy(dst_hbm.at[pl.ds(ebase + nch * _C, tail)],
                             didx_t, si_t)

        def fill_ones(i, _):
            ones[i, :] = ov
            return 0
        lax.fori_loop(0, _C, fill_ones, 0)

        def fill_zero(i, _):
            zbuf[i, :] = zv
            return 0
        lax.fori_loop(0, zr, fill_zero, 0)

        def zero_acc(i, _):
            pltpu.sync_copy(zbuf, acc.at[pl.ds(sid * rps + i * zr, zr)])
            return 0
        lax.fori_loop(0, rps // zr, zero_acc, 0)
        plsc.subcore_barrier()

        def wait_scatter(b):
            pltpu.make_async_copy(ones, acc.at[didx[b]], ss[b]).wait()

        def start(j, b):
            pl.when(j >= nb)(lambda: wait_scatter(b))
            pltpu.async_copy(dst_hbm.at[pl.ds(ebase + j * _C, _C)],
                             didx[b], si[b])

        def finish(j, b):
            del j
            pltpu.make_async_copy(dst_hbm.at[pl.ds(0, _C)],
                                  didx[b], si[b]).wait()
            pltpu.async_copy(ones, acc.at[didx[b]], ss[b], add=True)

        _pipeline(nch, start, finish, nb)
        for b in range(min(nb, nch)):
            wait_scatter(b)
        if tail:
            pltpu.make_async_copy(dst_hbm.at[pl.ds(0, tail)],
                                  didx_t, si_t).wait()
            pltpu.sync_copy(ones.at[pl.ds(0, tail)], acc.at[didx_t], add=True)
        plsc.subcore_barrier()

        pltpu.sync_copy(acc.at[pl.ds(sid * rps, rps)],
                        out_hbm.at[cid, pl.ds(sid * rps, rps)])

    return deg_k


@functools.lru_cache(maxsize=None)
def _agg_kernel(N, E, H):
    epw = E // _NW
    nch = epw // _C
    tail = epw - nch * _C
    npad, rps = _pad_rows(N)
    zr = 32
    assert rps % zr == 0 and tail % 8 == 0 and epw % 8 == 0
    mesh = plsc.VectorSubcoreMesh(core_axis_name="c", subcore_axis_name="s",
                                  num_cores=_NC, num_subcores=_NS)

    @functools.partial(
        pl.kernel,
        out_type=jax.ShapeDtypeStruct((_NC, npad, H), jnp.float32),
        mesh=mesh,
        scratch_types=[
            pltpu.VMEM((epw,), jnp.int32),
            [pltpu.VMEM((_C,), jnp.int32) for _ in range(_NB)],
            pltpu.VMEM((tail,), jnp.int32) if tail else None,
            [pltpu.VMEM((_C, H), jnp.float32) for _ in range(_NB)],
            pltpu.VMEM((zr, H), jnp.float32),
            pltpu.VMEM_SHARED((npad, H), jnp.float32),
            [pltpu.SemaphoreType.DMA for _ in range(_NB)],
            [pltpu.SemaphoreType.DMA for _ in range(_NB)],
            [pltpu.SemaphoreType.DMA for _ in range(_NB)],
            pltpu.SemaphoreType.DMA,
        ],
    )
    def agg_k(y_hbm, src_hbm, dst_hbm, out_hbm, sidx_all, didx, didx_t,
              rows, zbuf, acc, si, sg, ss, s_t):
        cid = lax.axis_index("c")
        sid = lax.axis_index("s")
        wid = sid * _NC + cid
        ebase = wid * epw
        zv = jnp.zeros((_L,), jnp.float32)

        pltpu.async_copy(src_hbm.at[pl.ds(ebase, epw)], sidx_all, s_t)
        if tail:
            pltpu.async_copy(dst_hbm.at[pl.ds(ebase + nch * _C, tail)],
                             didx_t, si[0])

        def fill_zero(i, _):
            for k in range(H // _L):
                zbuf[i, pl.ds(k * _L, _L)] = zv
            return 0
        lax.fori_loop(0, zr, fill_zero, 0)

        def zero_start(i, _):
            pltpu.async_copy(zbuf, acc.at[pl.ds(sid * rps + i * zr, zr)],
                             sg[0])
            return 0
        lax.fori_loop(0, rps // zr, zero_start, 0)

        def zero_wait(i, _):
            pltpu.make_async_copy(zbuf, acc.at[pl.ds(sid * rps + i * zr, zr)],
                                  sg[0]).wait()
            return 0
        lax.fori_loop(0, rps // zr, zero_wait, 0)
        pltpu.make_async_copy(src_hbm.at[pl.ds(0, epw)], sidx_all, s_t).wait()
        if tail:
            pltpu.make_async_copy(dst_hbm.at[pl.ds(0, tail)],
                                  didx_t, si[0]).wait()
        plsc.subcore_barrier()

        def wait_scatter(b):
            pltpu.make_async_copy(rows[b], acc.at[didx[b]], ss[b]).wait()

        def start(j, b):
            pl.when(j >= _NB)(lambda: wait_scatter(b))
            pltpu.async_copy(dst_hbm.at[pl.ds(ebase + j * _C, _C)],
                             didx[b], si[b])
            pltpu.async_copy(y_hbm.at[sidx_all.at[pl.ds(j * _C, _C)]],
                             rows[b], sg[b])

        def finish(j, b):
            del j
            pltpu.make_async_copy(dst_hbm.at[pl.ds(0, _C)],
                                  didx[b], si[b]).wait()
            pltpu.make_async_copy(y_hbm.at[pl.ds(0, _C)], rows[b],
                                  sg[b]).wait()
            pltpu.async_copy(rows[b], acc.at[didx[b]], ss[b], add=True)

        _pipeline(nch, start, finish, _NB)
        for b in range(min(_NB, nch)):
            wait_scatter(b)
        if tail:
            rows_t = rows[0].at[pl.ds(0, tail)]
            pltpu.async_copy(y_hbm.at[sidx_all.at[pl.ds(nch * _C, tail)]],
                             rows_t, s_t)
            pltpu.make_async_copy(y_hbm.at[pl.ds(0, tail)], rows_t,
                                  s_t).wait()
            pltpu.sync_copy(rows_t, acc.at[didx_t], add=True)
        plsc.subcore_barrier()

        pltpu.sync_copy(acc.at[pl.ds(sid * rps, rps)],
                        out_hbm.at[cid, pl.ds(sid * rps, rps)])

    return agg_k


def _dinv_block(dp_ref):
    deg = dp_ref[0, :, 0:1] + dp_ref[1, :, 0:1] + 1.0
    return lax.rsqrt(deg)


def _prep_call(deg_parts, x, W):
    N, H = x.shape
    bn = 2000

    def body(dp_ref, x_ref, w_ref, y_ref):
        dinv = _dinv_block(dp_ref)
        z = jnp.dot(x_ref[...], w_ref[...], preferred_element_type=jnp.float32)
        y_ref[...] = z * dinv

    return pl.pallas_call(
        body,
        grid=(N // bn,),
        in_specs=[
            pl.BlockSpec((_NC, bn, _DW), lambda i: (0, i, 0)),
            pl.BlockSpec((bn, H), lambda i: (i, 0)),
            pl.BlockSpec((H, H), lambda i: (0, 0)),
        ],
        out_specs=pl.BlockSpec((bn, H), lambda i: (i, 0)),
        out_shape=jax.ShapeDtypeStruct((N, H), jnp.float32),
    )(deg_parts, x, W)


def _mid_call(deg_parts, agg, y, b, W):
    N, H = y.shape
    bn = 2000

    def body(dp_ref, a_ref, y_ref, b_ref, w_ref, o_ref):
        dinv = _dinv_block(dp_ref)
        h = (a_ref[0] + a_ref[1] + y_ref[...]) * dinv + b_ref[...]
        o_ref[...] = jnp.dot(h, w_ref[...],
                             preferred_element_type=jnp.float32) * dinv

    return pl.pallas_call(
        body,
        grid=(N // bn,),
        in_specs=[
            pl.BlockSpec((_NC, bn, _DW), lambda i: (0, i, 0)),
            pl.BlockSpec((_NC, bn, H), lambda i: (0, i, 0)),
            pl.BlockSpec((bn, H), lambda i: (i, 0)),
            pl.BlockSpec((1, H), lambda i: (0, 0)),
            pl.BlockSpec((H, H), lambda i: (0, 0)),
        ],
        out_specs=pl.BlockSpec((bn, H), lambda i: (i, 0)),
        out_shape=jax.ShapeDtypeStruct((N, H), jnp.float32),
    )(deg_parts, agg, y, b, W)


def _fin_call(deg_parts, agg, y, b):
    N, H = y.shape
    bn = 2000

    def body(dp_ref, a_ref, y_ref, b_ref, o_ref):
        dinv = _dinv_block(dp_ref)
        o_ref[...] = (a_ref[0] + a_ref[1] + y_ref[...]) * dinv + b_ref[...]

    return pl.pallas_call(
        body,
        grid=(N // bn,),
        in_specs=[
            pl.BlockSpec((_NC, bn, _DW), lambda i: (0, i, 0)),
            pl.BlockSpec((_NC, bn, H), lambda i: (0, i, 0)),
            pl.BlockSpec((bn, H), lambda i: (i, 0)),
            pl.BlockSpec((1, H), lambda i: (0, 0)),
        ],
        out_specs=pl.BlockSpec((bn, H), lambda i: (i, 0)),
        out_shape=jax.ShapeDtypeStruct((N, H), jnp.float32),
    )(deg_parts, agg, y, b)


def kernel(embeddings, edge_index, W1, b1, W2, b2):
    x = embeddings
    src = edge_index[0].astype(jnp.int32)
    dst = edge_index[1].astype(jnp.int32)
    N, H = x.shape
    E = src.shape[0]

    deg_parts = _deg_kernel(N, E)(dst)
    y1 = _prep_call(deg_parts, x, W1)
    agg1 = _agg_kernel(N, E, H)(y1, src, dst)
    y2 = _mid_call(deg_parts, agg1, y1, b1.reshape(1, H), W2)
    agg2 = _agg_kernel(N, E, H)(y2, src, dst)
    return _fin_call(deg_parts, agg2, y2, b2.reshape(1, H))
